# baseline (speedup 1.0000x reference)
import jax
import jax.numpy as jnp
from jax import lax
from jax.experimental import pallas as pl
from jax.experimental.pallas import tpu as pltpu
from jax.experimental.pallas import tpu_sc as plsc

D = 64
SCALE = 8.0
NC = 2
NS = 16
NW = NC * NS
LANES = 16

V = 1000000
CB = 896
NBLK = V // CB
VTAIL = V - NBLK * CB

NBUF = 4
LOOKAHEAD = 2
C = 256


def _transpose_scale(tab_t, tail_lin):
    mesh = plsc.VectorSubcoreMesh(core_axis_name="c", subcore_axis_name="s")

    @pl.kernel(
        out_type=jax.ShapeDtypeStruct((V * D,), jnp.float32),
        mesh=mesh,
        scratch_types=[
            pltpu.VMEM((D, CB), jnp.float32),
            pltpu.VMEM((CB * D,), jnp.float32),
            pltpu.VMEM((VTAIL * D,), jnp.float32),
        ],
        compiler_params=pltpu.CompilerParams(
            use_tc_tiling_on_sc=True, needs_layout_passes=False),
    )
    def tkern(tab_hbm, tail_hbm, out_hbm, vbuf, obuf, tbuf):
        wid = lax.axis_index("s") * NC + lax.axis_index("c")
        n_w = jnp.where(wid < NBLK % NW, NBLK // NW + 1, NBLK // NW)

        lane = lax.iota(jnp.int32, 16)

        def blk_body(t, carry):
            c0 = (wid + t * NW) * CB
            c0 = pl.multiple_of(c0, 128)
            pltpu.sync_copy(tab_hbm.at[:, pl.ds(c0, CB)], vbuf)

            @plsc.parallel_loop(0, D * CB // LANES, 1, unroll=8)
            def _(t):
                v = t >> 2
                row = (t & 3) * LANES + lane
                col = jnp.broadcast_to(v, (LANES,))
                vals = plsc.load_gather(vbuf, [row, col]) * SCALE
                obuf[pl.ds(t * LANES, LANES)] = vals

            pltpu.sync_copy(obuf, out_hbm.at[pl.ds(c0 * D, CB * D)])
            return carry

        lax.fori_loop(0, n_w, blk_body, 0)

        @pl.when(wid == NW - 1)
        def _():
            pltpu.sync_copy(tail_hbm, tbuf)
            pltpu.sync_copy(tbuf, out_hbm.at[pl.ds(NBLK * CB * D, VTAIL * D)])

    return tkern(tab_t, tail_lin)


def _gather(idx_flat, table_lin):
    B = idx_flat.shape[0]
    per_w = B // NW
    n_chunks = per_w // C
    n_groups = n_chunks // NBUF

    mesh = plsc.VectorSubcoreMesh(core_axis_name="c", subcore_axis_name="s")

    @pl.kernel(
        out_type=jax.ShapeDtypeStruct((B, D), jnp.float32),
        mesh=mesh,
        scratch_types=(
            [pltpu.VMEM((per_w,), jnp.int32)]
            + [pltpu.VMEM((C, D), jnp.float32) for _ in range(NBUF)]
            + [pltpu.SemaphoreType.DMA for _ in range(NBUF)]
            + [pltpu.SemaphoreType.DMA for _ in range(NBUF)]
        ),
        compiler_params=pltpu.CompilerParams(use_tc_tiling_on_sc=False),
    )
    def gkern(idx_hbm, table_hbm, out_hbm, idx_v, *bufs_and_sems):
        bufs = bufs_and_sems[:NBUF]
        gsem = bufs_and_sems[NBUF:2 * NBUF]
        ssem = bufs_and_sems[2 * NBUF:3 * NBUF]

        wid = lax.axis_index("s") * NC + lax.axis_index("c")
        base = wid * per_w

        pltpu.sync_copy(idx_hbm.at[pl.ds(base, per_w)], idx_v)

        def issue_gather(g, b):
            pltpu.async_copy(
                table_hbm.at[idx_v.at[pl.ds(g * C, C)]], bufs[b], gsem[b])

        def wait_gather(g, b):
            pltpu.make_async_copy(
                table_hbm.at[idx_v.at[pl.ds(g * C, C)]], bufs[b],
                gsem[b]).wait()

        def issue_store(g, b):
            pltpu.async_copy(
                bufs[b], out_hbm.at[pl.ds(base + g * C, C)], ssem[b])

        def wait_store(g, b):
            pltpu.make_async_copy(
                bufs[b], out_hbm.at[pl.ds(base + g * C, C)], ssem[b]).wait()

        for g in range(LOOKAHEAD):
            issue_gather(g, g)

        def group_body(gi, carry):
            for p in range(NBUF):
                g = gi * NBUF + p
                q = (p + LOOKAHEAD) % NBUF
                wait_gather(g, p)
                issue_store(g, p)
                if p < NBUF - LOOKAHEAD:
                    @pl.when(gi >= 1)
                    def _():
                        wait_store(g - LOOKAHEAD, q)
                    issue_gather(g + LOOKAHEAD, q)
                else:
                    @pl.when(gi < n_groups - 1)
                    def _():
                        wait_store(g - LOOKAHEAD, q)
                        issue_gather(g + LOOKAHEAD, q)
            return carry

        lax.fori_loop(0, n_groups, group_body, 0)

        for p in range(NBUF):
            wait_store(n_chunks - NBUF + p, p)

    return gkern(idx_flat, table_lin)


def kernel(indices, table):
    B = indices.shape[0] * indices.shape[1]
    idx_flat = indices.reshape(B).astype(jnp.int32)

    tail_lin = (table[NBLK * CB:] * SCALE).reshape(VTAIL * D)
    tab_lin = _transpose_scale(table.T, tail_lin)
    out = _gather(idx_flat, tab_lin.reshape(V, D))
    return out.reshape(indices.shape[0], indices.shape[1], D)


# speedup vs baseline: 1.6130x; 1.6130x over previous
import jax
import jax.numpy as jnp
from jax import lax
from jax.experimental import pallas as pl
from jax.experimental.pallas import tpu as pltpu
from jax.experimental.pallas import tpu_sc as plsc

D = 64
SCALE = 8.0
NC = 2
NS = 16
NW = NC * NS
LANES = 16

V = 1000000
CB = 512
PAD = CB + 1
NBLK = V // CB
VTAIL = V - NBLK * CB

NBUF = 4
LOOKAHEAD = 2
C = 256


def _transpose_scale(tab_t, tail_lin):
    mesh = plsc.VectorSubcoreMesh(core_axis_name="c", subcore_axis_name="s")

    @pl.kernel(
        out_type=jax.ShapeDtypeStruct((V * D,), jnp.float32),
        mesh=mesh,
        scratch_types=[
            pltpu.VMEM((D, CB), jnp.float32),
            pltpu.VMEM((D * PAD,), jnp.float32),
            pltpu.VMEM((CB * D,), jnp.float32),
            pltpu.VMEM((VTAIL * D,), jnp.float32),
        ],
        compiler_params=pltpu.CompilerParams(
            use_tc_tiling_on_sc=True, needs_layout_passes=False),
    )
    def tkern(tab_hbm, tail_hbm, out_hbm, vbuf, pbuf, obuf, tbuf):
        wid = lax.axis_index("s") * NC + lax.axis_index("c")
        n_w = jnp.where(wid < NBLK % NW, NBLK // NW + 1, NBLK // NW)

        lane = lax.iota(jnp.int32, 16)

        def blk_body(t, carry):
            c0 = (wid + t * NW) * CB
            c0 = pl.multiple_of(c0, 128)
            pltpu.sync_copy(tab_hbm.at[:, pl.ds(c0, CB)], vbuf)

            @plsc.parallel_loop(0, D * CB // LANES, 1, unroll=8)
            def _(t):
                d = t & (D - 1)
                k = t >> 6
                row = jnp.broadcast_to(d, (LANES,))
                vals = plsc.load_gather(vbuf, [row, k * LANES + lane])
                plsc.store_scatter(pbuf, [d * PAD + k * LANES + lane], vals)

            @plsc.parallel_loop(0, CB * D // LANES, 1, unroll=8)
            def _(t):
                v = t >> 2
                j = t & 3
                addr = (j * LANES + lane) * PAD + v
                vals = plsc.load_gather(pbuf, [addr]) * SCALE
                obuf[pl.ds(t * LANES, LANES)] = vals

            pltpu.sync_copy(obuf, out_hbm.at[pl.ds(c0 * D, CB * D)])
            return carry

        lax.fori_loop(0, n_w, blk_body, 0)

        @pl.when(wid == NW - 1)
        def _():
            pltpu.sync_copy(tail_hbm, tbuf)
            pltpu.sync_copy(tbuf, out_hbm.at[pl.ds(NBLK * CB * D, VTAIL * D)])

    return tkern(tab_t, tail_lin)


def _gather(idx_flat, table_lin):
    B = idx_flat.shape[0]
    per_w = B // NW
    n_chunks = per_w // C
    n_groups = n_chunks // NBUF

    mesh = plsc.VectorSubcoreMesh(core_axis_name="c", subcore_axis_name="s")

    @pl.kernel(
        out_type=jax.ShapeDtypeStruct((B, D), jnp.float32),
        mesh=mesh,
        scratch_types=(
            [pltpu.VMEM((per_w,), jnp.int32)]
            + [pltpu.VMEM((C, D), jnp.float32) for _ in range(NBUF)]
            + [pltpu.SemaphoreType.DMA for _ in range(NBUF)]
            + [pltpu.SemaphoreType.DMA for _ in range(NBUF)]
        ),
        compiler_params=pltpu.CompilerParams(use_tc_tiling_on_sc=False),
    )
    def gkern(idx_hbm, table_hbm, out_hbm, idx_v, *bufs_and_sems):
        bufs = bufs_and_sems[:NBUF]
        gsem = bufs_and_sems[NBUF:2 * NBUF]
        ssem = bufs_and_sems[2 * NBUF:3 * NBUF]

        wid = lax.axis_index("s") * NC + lax.axis_index("c")
        base = wid * per_w

        pltpu.sync_copy(idx_hbm.at[pl.ds(base, per_w)], idx_v)

        def issue_gather(g, b):
            pltpu.async_copy(
                table_hbm.at[idx_v.at[pl.ds(g * C, C)]], bufs[b], gsem[b])

        def wait_gather(g, b):
            pltpu.make_async_copy(
                table_hbm.at[idx_v.at[pl.ds(g * C, C)]], bufs[b],
                gsem[b]).wait()

        def issue_store(g, b):
            pltpu.async_copy(
                bufs[b], out_hbm.at[pl.ds(base + g * C, C)], ssem[b])

        def wait_store(g, b):
            pltpu.make_async_copy(
                bufs[b], out_hbm.at[pl.ds(base + g * C, C)], ssem[b]).wait()

        for g in range(LOOKAHEAD):
            issue_gather(g, g)

        def group_body(gi, carry):
            for p in range(NBUF):
                g = gi * NBUF + p
                q = (p + LOOKAHEAD) % NBUF
                wait_gather(g, p)
                issue_store(g, p)
                if p < NBUF - LOOKAHEAD:
                    @pl.when(gi >= 1)
                    def _():
                        wait_store(g - LOOKAHEAD, q)
                    issue_gather(g + LOOKAHEAD, q)
                else:
                    @pl.when(gi < n_groups - 1)
                    def _():
                        wait_store(g - LOOKAHEAD, q)
                        issue_gather(g + LOOKAHEAD, q)
            return carry

        lax.fori_loop(0, n_groups, group_body, 0)

        for p in range(NBUF):
            wait_store(n_chunks - NBUF + p, p)

    return gkern(idx_flat, table_lin)


def kernel(indices, table):
    B = indices.shape[0] * indices.shape[1]
    idx_flat = indices.reshape(B).astype(jnp.int32)

    tail_lin = (table[NBLK * CB:] * SCALE).reshape(VTAIL * D)
    tab_lin = _transpose_scale(table.T, tail_lin)
    out = _gather(idx_flat, tab_lin.reshape(V, D))
    return out.reshape(indices.shape[0], indices.shape[1], D)


# speedup vs baseline: 1.9698x; 1.2213x over previous
import jax
import jax.numpy as jnp
from jax import lax
from jax.experimental import pallas as pl
from jax.experimental.pallas import tpu as pltpu
from jax.experimental.pallas import tpu_sc as plsc

D = 64
SCALE = 8.0
NC = 2
NS = 16
NW = NC * NS
LANES = 16

V = 1000000
CB = 384
PAD = CB + 1
NBLK = V // CB
VTAIL = V - NBLK * CB

NBUF = 4
LOOKAHEAD = 2
C = 256


def _transpose_scale(tab_t, tail_lin):
    mesh = plsc.VectorSubcoreMesh(core_axis_name="c", subcore_axis_name="s")

    @pl.kernel(
        out_type=jax.ShapeDtypeStruct((V * D,), jnp.float32),
        mesh=mesh,
        scratch_types=[
            pltpu.VMEM((D, CB), jnp.float32),
            pltpu.VMEM((D, CB), jnp.float32),
            pltpu.VMEM((D * PAD,), jnp.float32),
            pltpu.VMEM((CB * D,), jnp.float32),
            pltpu.VMEM((CB * D,), jnp.float32),
            pltpu.VMEM((VTAIL * D,), jnp.float32),
            pltpu.SemaphoreType.DMA,
            pltpu.SemaphoreType.DMA,
            pltpu.SemaphoreType.DMA,
            pltpu.SemaphoreType.DMA,
        ],
        compiler_params=pltpu.CompilerParams(
            use_tc_tiling_on_sc=True, needs_layout_passes=False),
    )
    def tkern(tab_hbm, tail_hbm, out_hbm, vbuf0, vbuf1, pbuf, obuf0, obuf1,
              tbuf, gs0, gs1, ss0, ss1):
        wid = lax.axis_index("s") * NC + lax.axis_index("c")
        n_w = jnp.where(wid < NBLK % NW, NBLK // NW + 1, NBLK // NW)

        lane = lax.iota(jnp.int32, 16)
        vbufs, obufs = (vbuf0, vbuf1), (obuf0, obuf1)
        gsems, ssems = (gs0, gs1), (ss0, ss1)

        def col0(t):
            c0 = (wid + t * NW) * CB
            return pl.multiple_of(c0, 128)

        def issue_in(t, b):
            pltpu.async_copy(tab_hbm.at[:, pl.ds(col0(t), CB)], vbufs[b],
                             gsems[b])

        def wait_in(t, b):
            pltpu.make_async_copy(tab_hbm.at[:, pl.ds(col0(t), CB)],
                                  vbufs[b], gsems[b]).wait()

        def issue_out(t, b):
            pltpu.async_copy(obufs[b],
                             out_hbm.at[pl.ds(col0(t) * D, CB * D)], ssems[b])

        def wait_out(t, b):
            pltpu.make_async_copy(obufs[b],
                                  out_hbm.at[pl.ds(col0(t) * D, CB * D)],
                                  ssems[b]).wait()

        def transpose_block(b):
            vbuf, obuf = vbufs[b], obufs[b]

            @plsc.parallel_loop(0, D * CB // LANES, 1, unroll=8)
            def _(t):
                d = t & (D - 1)
                k = t >> 6
                row = jnp.broadcast_to(d, (LANES,))
                vals = plsc.load_gather(vbuf, [row, k * LANES + lane])
                plsc.store_scatter(pbuf, [d * PAD + k * LANES + lane], vals)

            for j in range(4):
                base = j * LANES * PAD
                addr0 = lane * PAD + base

                @plsc.parallel_loop(0, CB, 1, unroll=8)
                def _(v):
                    vals = plsc.load_gather(pbuf, [addr0 + v]) * SCALE
                    obuf[pl.ds(v * D + j * LANES, LANES)] = vals

        @pl.when(n_w > 0)
        def _():
            issue_in(0, 0)

            def blk_body(t, carry):
                b = lax.rem(t, 2)

                def run(b):
                    wait_in(t, b)

                    @pl.when(t + 1 < n_w)
                    def _():
                        issue_in(t + 1, 1 - b)

                    transpose_block(b)

                    @pl.when(t >= 2)
                    def _():
                        wait_out(t - 2, b)

                    issue_out(t, b)

                lax.cond(b == 0, lambda: run(0), lambda: run(1))
                return carry

            lax.fori_loop(0, n_w, blk_body, 0)

            def drain(t):
                lax.cond(lax.rem(t, 2) == 0,
                         lambda: wait_out(t, 0), lambda: wait_out(t, 1))

            @pl.when(n_w >= 2)
            def _():
                drain(n_w - 2)
            drain(n_w - 1)

        @pl.when(wid == NW - 1)
        def _():
            pltpu.sync_copy(tail_hbm, tbuf)
            pltpu.sync_copy(tbuf, out_hbm.at[pl.ds(NBLK * CB * D, VTAIL * D)])

    return tkern(tab_t, tail_lin)


def _gather(idx_flat, table_lin):
    B = idx_flat.shape[0]
    per_w = B // NW
    n_chunks = per_w // C
    n_groups = n_chunks // NBUF

    mesh = plsc.VectorSubcoreMesh(core_axis_name="c", subcore_axis_name="s")

    @pl.kernel(
        out_type=jax.ShapeDtypeStruct((B, D), jnp.float32),
        mesh=mesh,
        scratch_types=(
            [pltpu.VMEM((per_w,), jnp.int32)]
            + [pltpu.VMEM((C, D), jnp.float32) for _ in range(NBUF)]
            + [pltpu.SemaphoreType.DMA for _ in range(NBUF)]
            + [pltpu.SemaphoreType.DMA for _ in range(NBUF)]
        ),
        compiler_params=pltpu.CompilerParams(use_tc_tiling_on_sc=False),
    )
    def gkern(idx_hbm, table_hbm, out_hbm, idx_v, *bufs_and_sems):
        bufs = bufs_and_sems[:NBUF]
        gsem = bufs_and_sems[NBUF:2 * NBUF]
        ssem = bufs_and_sems[2 * NBUF:3 * NBUF]

        wid = lax.axis_index("s") * NC + lax.axis_index("c")
        base = wid * per_w

        pltpu.sync_copy(idx_hbm.at[pl.ds(base, per_w)], idx_v)

        def issue_gather(g, b):
            pltpu.async_copy(
                table_hbm.at[idx_v.at[pl.ds(g * C, C)]], bufs[b], gsem[b])

        def wait_gather(g, b):
            pltpu.make_async_copy(
                table_hbm.at[idx_v.at[pl.ds(g * C, C)]], bufs[b],
                gsem[b]).wait()

        def issue_store(g, b):
            pltpu.async_copy(
                bufs[b], out_hbm.at[pl.ds(base + g * C, C)], ssem[b])

        def wait_store(g, b):
            pltpu.make_async_copy(
                bufs[b], out_hbm.at[pl.ds(base + g * C, C)], ssem[b]).wait()

        for g in range(LOOKAHEAD):
            issue_gather(g, g)

        def group_body(gi, carry):
            for p in range(NBUF):
                g = gi * NBUF + p
                q = (p + LOOKAHEAD) % NBUF
                wait_gather(g, p)
                issue_store(g, p)
                if p < NBUF - LOOKAHEAD:
                    @pl.when(gi >= 1)
                    def _():
                        wait_store(g - LOOKAHEAD, q)
                    issue_gather(g + LOOKAHEAD, q)
                else:
                    @pl.when(gi < n_groups - 1)
                    def _():
                        wait_store(g - LOOKAHEAD, q)
                        issue_gather(g + LOOKAHEAD, q)
            return carry

        lax.fori_loop(0, n_groups, group_body, 0)

        for p in range(NBUF):
            wait_store(n_chunks - NBUF + p, p)

    return gkern(idx_flat, table_lin)


def kernel(indices, table):
    B = indices.shape[0] * indices.shape[1]
    idx_flat = indices.reshape(B).astype(jnp.int32)

    tail_lin = (table[NBLK * CB:] * SCALE).reshape(VTAIL * D)
    tab_lin = _transpose_scale(table.T, tail_lin)
    out = _gather(idx_flat, tab_lin.reshape(V, D))
    return out.reshape(indices.shape[0], indices.shape[1], D)


# speedup vs baseline: 4.1237x; 2.0934x over previous
import jax
import jax.numpy as jnp
from jax import lax
from jax.experimental import pallas as pl
from jax.experimental.pallas import tpu as pltpu
from jax.experimental.pallas import tpu_sc as plsc

D = 64
SCALE = 8.0
NC = 2
NS = 16
NW = NC * NS
LANES = 16

V = 1000000
CB = 384
PAD = CB + 1
NBLK = V // CB
VTAIL = V - NBLK * CB

NBUF = 4
LOOKAHEAD = 2
C = 256


def _transpose_scale(tab_t, tail_lin):
    mesh = plsc.VectorSubcoreMesh(core_axis_name="c", subcore_axis_name="s")

    @pl.kernel(
        out_type=jax.ShapeDtypeStruct((V * D,), jnp.float32),
        mesh=mesh,
        scratch_types=[
            pltpu.VMEM((D, CB), jnp.float32),
            pltpu.VMEM((D, CB), jnp.float32),
            pltpu.VMEM((D * PAD,), jnp.float32),
            pltpu.VMEM((CB * D,), jnp.float32),
            pltpu.VMEM((CB * D,), jnp.float32),
            pltpu.VMEM((VTAIL * D,), jnp.float32),
            pltpu.SemaphoreType.DMA,
            pltpu.SemaphoreType.DMA,
            pltpu.SemaphoreType.DMA,
            pltpu.SemaphoreType.DMA,
        ],
        compiler_params=pltpu.CompilerParams(
            use_tc_tiling_on_sc=True, needs_layout_passes=False),
    )
    def tkern(tab_hbm, tail_hbm, out_hbm, vbuf0, vbuf1, pbuf, obuf0, obuf1,
              tbuf, gs0, gs1, ss0, ss1):
        wid = lax.axis_index("s") * NC + lax.axis_index("c")
        n_w = jnp.where(wid < NBLK % NW, NBLK // NW + 1, NBLK // NW)

        lane = lax.iota(jnp.int32, 16)
        vbufs, obufs = (vbuf0, vbuf1), (obuf0, obuf1)
        gsems, ssems = (gs0, gs1), (ss0, ss1)

        def col0(t):
            c0 = (wid + t * NW) * CB
            return pl.multiple_of(c0, 128)

        def issue_in(t, b):
            pltpu.async_copy(tab_hbm.at[:, pl.ds(col0(t), CB)], vbufs[b],
                             gsems[b])

        def wait_in(t, b):
            pltpu.make_async_copy(tab_hbm.at[:, pl.ds(col0(t), CB)],
                                  vbufs[b], gsems[b]).wait()

        def issue_out(t, b):
            pltpu.async_copy(obufs[b],
                             out_hbm.at[pl.ds(col0(t) * D, CB * D)], ssems[b])

        def wait_out(t, b):
            pltpu.make_async_copy(obufs[b],
                                  out_hbm.at[pl.ds(col0(t) * D, CB * D)],
                                  ssems[b]).wait()

        def transpose_block(b):
            vbuf, obuf = vbufs[b], obufs[b]

            @plsc.parallel_loop(0, D * CB // LANES, 1, unroll=8)
            def _(t):
                d = t & (D - 1)
                k = t >> 6
                vals = vbuf[d, pl.ds(k * LANES, LANES)]
                plsc.store_scatter(pbuf, [d * PAD + k * LANES + lane], vals)

            for j in range(4):
                base = j * LANES * PAD
                addr0 = lane * PAD + base

                @plsc.parallel_loop(0, CB, 1, unroll=8)
                def _(v):
                    vals = plsc.load_gather(pbuf, [addr0 + v]) * SCALE
                    obuf[pl.ds(v * D + j * LANES, LANES)] = vals

        @pl.when(n_w > 0)
        def _():
            issue_in(0, 0)

            def blk_body(t, carry):
                b = lax.rem(t, 2)

                def run(b):
                    wait_in(t, b)

                    @pl.when(t + 1 < n_w)
                    def _():
                        issue_in(t + 1, 1 - b)

                    transpose_block(b)

                    @pl.when(t >= 2)
                    def _():
                        wait_out(t - 2, b)

                    issue_out(t, b)

                lax.cond(b == 0, lambda: run(0), lambda: run(1))
                return carry

            lax.fori_loop(0, n_w, blk_body, 0)

            def drain(t):
                lax.cond(lax.rem(t, 2) == 0,
                         lambda: wait_out(t, 0), lambda: wait_out(t, 1))

            @pl.when(n_w >= 2)
            def _():
                drain(n_w - 2)
            drain(n_w - 1)

        @pl.when(wid == NW - 1)
        def _():
            pltpu.sync_copy(tail_hbm, tbuf)
            pltpu.sync_copy(tbuf, out_hbm.at[pl.ds(NBLK * CB * D, VTAIL * D)])

    return tkern(tab_t, tail_lin)


def _gather(idx_flat, table_lin):
    B = idx_flat.shape[0]
    per_w = B // NW
    CH = 128
    n_chunks = per_w // CH
    n_groups = n_chunks // NBUF
    TPAD = 65

    mesh = plsc.VectorSubcoreMesh(core_axis_name="c", subcore_axis_name="s")

    @pl.kernel(
        out_type=jax.ShapeDtypeStruct((B * D,), jnp.float32),
        mesh=mesh,
        scratch_types=(
            [pltpu.VMEM((per_w,), jnp.int32),
             pltpu.VMEM((CH * TPAD,), jnp.float32)]
            + [pltpu.VMEM((CH, D), jnp.float32) for _ in range(NBUF)]
            + [pltpu.VMEM((CH * D,), jnp.float32) for _ in range(NBUF)]
            + [pltpu.SemaphoreType.DMA for _ in range(NBUF)]
            + [pltpu.SemaphoreType.DMA for _ in range(NBUF)]
        ),
        compiler_params=pltpu.CompilerParams(
            use_tc_tiling_on_sc=False, needs_layout_passes=False),
    )
    def gkern(idx_hbm, table_hbm, out_hbm, idx_v, pbuf, *bufs_and_sems):
        gbufs = bufs_and_sems[:NBUF]
        obufs = bufs_and_sems[NBUF:2 * NBUF]
        gsem = bufs_and_sems[2 * NBUF:3 * NBUF]
        ssem = bufs_and_sems[3 * NBUF:4 * NBUF]

        wid = lax.axis_index("s") * NC + lax.axis_index("c")
        base = wid * per_w
        lane = lax.iota(jnp.int32, 16)

        pltpu.sync_copy(idx_hbm.at[pl.ds(base, per_w)], idx_v)

        def out_off(g, db):
            c = wid * n_chunks + g
            s = c >> 5
            bb = c & 31
            return pl.multiple_of((((s << 3) + db) << 5 | bb) << 10, 1024)

        def issue_gather(g, b):
            pltpu.async_copy(
                table_hbm.at[idx_v.at[pl.ds(g * CH, CH)]], gbufs[b], gsem[b])

        def wait_gather(g, b):
            pltpu.make_async_copy(
                table_hbm.at[idx_v.at[pl.ds(g * CH, CH)]], gbufs[b],
                gsem[b]).wait()

        def issue_store(g, b):
            for db in range(8):
                pltpu.async_copy(
                    obufs[b].at[pl.ds(db * 1024, 1024)],
                    out_hbm.at[pl.ds(out_off(g, db), 1024)], ssem[b])

        def wait_store(g, b):
            for db in range(8):
                pltpu.make_async_copy(
                    obufs[b].at[pl.ds(db * 1024, 1024)],
                    out_hbm.at[pl.ds(out_off(g, db), 1024)], ssem[b]).wait()

        def transpose_chunk(b):
            gbuf, obuf = gbufs[b], obufs[b]

            @plsc.parallel_loop(0, CH * (D // LANES), 1, unroll=8)
            def _(t):
                r = t >> 2
                j = t & 3
                vals = gbuf[r, pl.ds(j * LANES, LANES)]
                plsc.store_scatter(pbuf, [r * TPAD + j * LANES + lane], vals)

            @plsc.parallel_loop(0, D * (CH // LANES), 1, unroll=8)
            def _(t):
                d = t >> 3
                m = t & 7
                addr = lane * TPAD + (m * LANES * TPAD + d)
                obuf[pl.ds(t * LANES, LANES)] = plsc.load_gather(pbuf, [addr])

        for g in range(LOOKAHEAD):
            issue_gather(g, g)

        def group_body(gi, carry):
            for p in range(NBUF):
                g = gi * NBUF + p
                q = (p + LOOKAHEAD) % NBUF
                wait_gather(g, p)
                transpose_chunk(p)
                issue_store(g, p)
                if p < NBUF - LOOKAHEAD:
                    @pl.when(gi >= 1)
                    def _():
                        wait_store(g - LOOKAHEAD, q)
                    issue_gather(g + LOOKAHEAD, q)
                else:
                    @pl.when(gi < n_groups - 1)
                    def _():
                        wait_store(g - LOOKAHEAD, q)
                        issue_gather(g + LOOKAHEAD, q)
            return carry

        lax.fori_loop(0, n_groups, group_body, 0)

        for p in range(NBUF):
            wait_store(n_chunks - NBUF + p, p)

    return gkern(idx_flat, table_lin)


def kernel(indices, table):
    NB, NS_ = indices.shape
    B = NB * NS_
    idx_t = indices.T.reshape(B).astype(jnp.int32)

    tail_lin = (table[NBLK * CB:] * SCALE).reshape(VTAIL * D)
    tab_lin = _transpose_scale(table.T, tail_lin)
    out = _gather(idx_t, tab_lin.reshape(V, D))
    out5 = out.reshape(NS_, 8, NB // 128, 8, 128)
    return out5.transpose(2, 4, 0, 1, 3).reshape(NB, NS_, D)


# speedup vs baseline: 4.1717x; 1.0116x over previous
import jax
import jax.numpy as jnp
from jax import lax
from jax.experimental import pallas as pl
from jax.experimental.pallas import tpu as pltpu
from jax.experimental.pallas import tpu_sc as plsc

D = 64
SCALE = 8.0
NC = 2
NS = 16
NW = NC * NS
LANES = 16

V = 1000000
CB = 384
PAD = CB + 1
NBLK = V // CB
VTAIL = V - NBLK * CB

NBUF = 4
LOOKAHEAD = 2
C = 256


def _transpose_scale(tab_t, tail_lin):
    mesh = plsc.VectorSubcoreMesh(core_axis_name="c", subcore_axis_name="s")

    @pl.kernel(
        out_type=jax.ShapeDtypeStruct((V * D,), jnp.float32),
        mesh=mesh,
        scratch_types=[
            pltpu.VMEM((D, CB), jnp.float32),
            pltpu.VMEM((D, CB), jnp.float32),
            pltpu.VMEM((D * PAD,), jnp.float32),
            pltpu.VMEM((CB * D,), jnp.float32),
            pltpu.VMEM((CB * D,), jnp.float32),
            pltpu.VMEM((VTAIL * D,), jnp.float32),
            pltpu.SemaphoreType.DMA,
            pltpu.SemaphoreType.DMA,
            pltpu.SemaphoreType.DMA,
            pltpu.SemaphoreType.DMA,
        ],
        compiler_params=pltpu.CompilerParams(
            use_tc_tiling_on_sc=True, needs_layout_passes=False),
    )
    def tkern(tab_hbm, tail_hbm, out_hbm, vbuf0, vbuf1, pbuf, obuf0, obuf1,
              tbuf, gs0, gs1, ss0, ss1):
        wid = lax.axis_index("s") * NC + lax.axis_index("c")
        n_w = jnp.where(wid < NBLK % NW, NBLK // NW + 1, NBLK // NW)

        lane = lax.iota(jnp.int32, 16)
        vbufs, obufs = (vbuf0, vbuf1), (obuf0, obuf1)
        gsems, ssems = (gs0, gs1), (ss0, ss1)

        def col0(t):
            c0 = (wid + t * NW) * CB
            return pl.multiple_of(c0, 128)

        def issue_in(t, b):
            pltpu.async_copy(tab_hbm.at[:, pl.ds(col0(t), CB)], vbufs[b],
                             gsems[b])

        def wait_in(t, b):
            pltpu.make_async_copy(tab_hbm.at[:, pl.ds(col0(t), CB)],
                                  vbufs[b], gsems[b]).wait()

        def issue_out(t, b):
            pltpu.async_copy(obufs[b],
                             out_hbm.at[pl.ds(col0(t) * D, CB * D)], ssems[b])

        def wait_out(t, b):
            pltpu.make_async_copy(obufs[b],
                                  out_hbm.at[pl.ds(col0(t) * D, CB * D)],
                                  ssems[b]).wait()

        def transpose_block(b):
            vbuf, obuf = vbufs[b], obufs[b]

            @plsc.parallel_loop(0, D * CB // LANES, 1, unroll=16)
            def _(t):
                d = t & (D - 1)
                k = t >> 6
                vals = vbuf[d, pl.ds(k * LANES, LANES)]
                plsc.store_scatter(pbuf, [d * PAD + k * LANES + lane], vals)

            for j in range(4):
                base = j * LANES * PAD
                addr0 = lane * PAD + base

                @plsc.parallel_loop(0, CB, 1, unroll=16)
                def _(v):
                    vals = plsc.load_gather(pbuf, [addr0 + v]) * SCALE
                    obuf[pl.ds(v * D + j * LANES, LANES)] = vals

        @pl.when(n_w > 0)
        def _():
            issue_in(0, 0)

            def blk_body(t, carry):
                b = lax.rem(t, 2)

                def run(b):
                    wait_in(t, b)

                    @pl.when(t + 1 < n_w)
                    def _():
                        issue_in(t + 1, 1 - b)

                    transpose_block(b)

                    @pl.when(t >= 2)
                    def _():
                        wait_out(t - 2, b)

                    issue_out(t, b)

                lax.cond(b == 0, lambda: run(0), lambda: run(1))
                return carry

            lax.fori_loop(0, n_w, blk_body, 0)

            def drain(t):
                lax.cond(lax.rem(t, 2) == 0,
                         lambda: wait_out(t, 0), lambda: wait_out(t, 1))

            @pl.when(n_w >= 2)
            def _():
                drain(n_w - 2)
            drain(n_w - 1)

        @pl.when(wid == NW - 1)
        def _():
            pltpu.sync_copy(tail_hbm, tbuf)
            pltpu.sync_copy(tbuf, out_hbm.at[pl.ds(NBLK * CB * D, VTAIL * D)])

    return tkern(tab_t, tail_lin)


def _gather(idx_flat, table_lin):
    B = idx_flat.shape[0]
    per_w = B // NW
    CH = 128
    n_chunks = per_w // CH
    n_groups = n_chunks // NBUF
    TPAD = 65

    mesh = plsc.VectorSubcoreMesh(core_axis_name="c", subcore_axis_name="s")

    @pl.kernel(
        out_type=jax.ShapeDtypeStruct((B * D,), jnp.float32),
        mesh=mesh,
        scratch_types=(
            [pltpu.VMEM((per_w,), jnp.int32),
             pltpu.VMEM((CH * TPAD,), jnp.float32)]
            + [pltpu.VMEM((CH, D), jnp.float32) for _ in range(NBUF)]
            + [pltpu.VMEM((CH * D,), jnp.float32) for _ in range(NBUF)]
            + [pltpu.SemaphoreType.DMA for _ in range(NBUF)]
            + [pltpu.SemaphoreType.DMA for _ in range(NBUF)]
        ),
        compiler_params=pltpu.CompilerParams(
            use_tc_tiling_on_sc=False, needs_layout_passes=False),
    )
    def gkern(idx_hbm, table_hbm, out_hbm, idx_v, pbuf, *bufs_and_sems):
        gbufs = bufs_and_sems[:NBUF]
        obufs = bufs_and_sems[NBUF:2 * NBUF]
        gsem = bufs_and_sems[2 * NBUF:3 * NBUF]
        ssem = bufs_and_sems[3 * NBUF:4 * NBUF]

        wid = lax.axis_index("s") * NC + lax.axis_index("c")
        base = wid * per_w
        lane = lax.iota(jnp.int32, 16)

        pltpu.sync_copy(idx_hbm.at[pl.ds(base, per_w)], idx_v)

        def out_off(g, db):
            c = wid * n_chunks + g
            s = c >> 5
            bb = c & 31
            return pl.multiple_of((((s << 3) + db) << 5 | bb) << 10, 1024)

        def issue_gather(g, b):
            pltpu.async_copy(
                table_hbm.at[idx_v.at[pl.ds(g * CH, CH)]], gbufs[b], gsem[b])

        def wait_gather(g, b):
            pltpu.make_async_copy(
                table_hbm.at[idx_v.at[pl.ds(g * CH, CH)]], gbufs[b],
                gsem[b]).wait()

        def issue_store(g, b):
            for db in range(8):
                pltpu.async_copy(
                    obufs[b].at[pl.ds(db * 1024, 1024)],
                    out_hbm.at[pl.ds(out_off(g, db), 1024)], ssem[b])

        def wait_store(g, b):
            for db in range(8):
                pltpu.make_async_copy(
                    obufs[b].at[pl.ds(db * 1024, 1024)],
                    out_hbm.at[pl.ds(out_off(g, db), 1024)], ssem[b]).wait()

        def transpose_chunk(b):
            gbuf, obuf = gbufs[b], obufs[b]

            @plsc.parallel_loop(0, CH * (D // LANES), 1, unroll=16)
            def _(t):
                r = t >> 2
                j = t & 3
                vals = gbuf[r, pl.ds(j * LANES, LANES)]
                plsc.store_scatter(pbuf, [r * TPAD + j * LANES + lane], vals)

            @plsc.parallel_loop(0, D * (CH // LANES), 1, unroll=16)
            def _(t):
                d = t >> 3
                m = t & 7
                addr = lane * TPAD + (m * LANES * TPAD + d)
                obuf[pl.ds(t * LANES, LANES)] = plsc.load_gather(pbuf, [addr])

        for g in range(LOOKAHEAD):
            issue_gather(g, g)

        def group_body(gi, carry):
            for p in range(NBUF):
                g = gi * NBUF + p
                q = (p + LOOKAHEAD) % NBUF
                wait_gather(g, p)
                transpose_chunk(p)
                issue_store(g, p)
                if p < NBUF - LOOKAHEAD:
                    @pl.when(gi >= 1)
                    def _():
                        wait_store(g - LOOKAHEAD, q)
                    issue_gather(g + LOOKAHEAD, q)
                else:
                    @pl.when(gi < n_groups - 1)
                    def _():
                        wait_store(g - LOOKAHEAD, q)
                        issue_gather(g + LOOKAHEAD, q)
            return carry

        lax.fori_loop(0, n_groups, group_body, 0)

        for p in range(NBUF):
            wait_store(n_chunks - NBUF + p, p)

    return gkern(idx_flat, table_lin)


def kernel(indices, table):
    NB, NS_ = indices.shape
    B = NB * NS_
    idx_t = indices.T.reshape(B).astype(jnp.int32)

    tail_lin = (table[NBLK * CB:] * SCALE).reshape(VTAIL * D)
    tab_lin = _transpose_scale(table.T, tail_lin)
    out = _gather(idx_t, tab_lin.reshape(V, D))
    out5 = out.reshape(NS_, 8, NB // 128, 8, 128)
    return out5.transpose(2, 4, 0, 1, 3).reshape(NB, NS_, D)
